# baseline (speedup 1.0000x reference)
import functools

import jax
import jax.numpy as jnp
from jax.experimental import pallas as pl
from jax.experimental.pallas import tpu as pltpu

EPS = 1e-5
_B = 16384

_DN = (((1,), (0,)), ((), ()))
_AXIS = "core"


def _make_kernel(n, nb):
    def _kernel(xyzT_hbm, featT_hbm, m_hbm, w_hbm, g_hbm, b_hbm,
                outT_hbm, velT_hbm, parts_hbm,
                stash_ref, w_ref, gb_ref, s_ref, q_ref, sq_ref, pp_ref,
                sc_ref, sh_ref, c1_ref, c2_ref, sem):
        ncores = jax.lax.axis_size(_AXIS)
        cid = jax.lax.axis_index(_AXIS)
        pltpu.sync_copy(w_hbm, w_ref)
        pltpu.sync_copy(g_hbm, gb_ref.at[0:1])
        pltpu.sync_copy(b_hbm, gb_ref.at[1:2])
        s_ref[...] = jnp.zeros_like(s_ref)
        q_ref[...] = jnp.zeros_like(q_ref)
        c1_ref[0] = 0
        c2_ref[0] = 0

        def _stage0(a_ref, f_ref, m_ref, vel_ref):
            i = c1_ref[0]
            a = a_ref[...]
            f = f_ref[...]
            vel_ref[...] = f
            w3 = w_ref[:, 0:3]
            wf = w_ref[:, 3:16]
            p = jax.lax.dot_general(w3, a, _DN,
                                    preferred_element_type=jnp.float32)
            p = p + jax.lax.dot_general(wf, f, _DN,
                                        preferred_element_type=jnp.float32)
            stash_ref[i] = p.astype(jnp.bfloat16)
            pm = jnp.where(m_ref[...] > 0.0, p, 0.0)
            s_ref[...] += jnp.sum(pm, axis=1, keepdims=True)
            q_ref[...] += jnp.sum(pm * pm, axis=1, keepdims=True)
            c1_ref[0] = i + 1

        row = lambda i: (0, i)
        pltpu.emit_pipeline(
            _stage0,
            grid=(nb,),
            in_specs=[
                pl.BlockSpec((3, _B), row),
                pl.BlockSpec((13, _B), row),
                pl.BlockSpec((1, _B), row),
            ],
            out_specs=[pl.BlockSpec((13, _B), row)],
            core_axis_name=_AXIS,
        )(xyzT_hbm, featT_hbm, m_hbm, velT_hbm)

        sq_ref[0:1] = s_ref[...][None]
        sq_ref[1:2] = q_ref[...][None]
        pltpu.sync_copy(sq_ref, parts_hbm.at[cid])

        @pl.when(ncores > 1)
        def _():
            def signal(i):
                @pl.when(cid != i)
                def _():
                    pl.semaphore_signal(sem, 1, core_index=i)
            for i in range(2):
                @pl.when(i < ncores)
                def _():
                    signal(i)
            pl.semaphore_wait(sem, ncores - 1)

        pltpu.sync_copy(parts_hbm, pp_ref)
        tot = jnp.sum(pp_ref[...], axis=0)
        mean = tot[0] / n
        var = tot[1] / n - mean * mean
        scale = gb_ref[0] * jax.lax.rsqrt(var + EPS)
        sc_ref[...] = scale
        sh_ref[...] = gb_ref[1] - mean * scale

        def _stage1(out_ref):
            j = c2_ref[0]
            p = stash_ref[j].astype(jnp.float32)
            out_ref[...] = jnp.maximum(p * sc_ref[...] + sh_ref[...], 0.0)
            c2_ref[0] = j + 1

        pltpu.emit_pipeline(
            _stage1,
            grid=(nb,),
            out_specs=[pl.BlockSpec((16, _B), row)],
            core_axis_name=_AXIS,
        )(outT_hbm)

    return _kernel


def kernel(xyz, feat, offset, velocities, W, gamma, beta):
    n = xyz.shape[0]
    nb = pl.cdiv(n, _B)
    mask = (jnp.arange(nb * _B, dtype=jnp.int32) < n
            ).astype(jnp.float32).reshape(1, nb * _B)
    xyzT = xyz.T
    featT = feat.T
    g = gamma.reshape(1, 16, 1)
    b = beta.reshape(1, 16, 1)

    mesh = pltpu.create_tensorcore_mesh(_AXIS)

    fn = pl.kernel(
        _make_kernel(float(n), nb),
        out_type=[
            jax.ShapeDtypeStruct((16, n), jnp.float32),
            jax.ShapeDtypeStruct((13, n), jnp.float32),
            jax.ShapeDtypeStruct((2, 2, 16, 1), jnp.float32),
        ],
        mesh=mesh,
        scratch_types=[
            pltpu.VMEM((nb, 16, _B), jnp.bfloat16),
            pltpu.VMEM((16, 16), jnp.float32),
            pltpu.VMEM((2, 16, 1), jnp.float32),
            pltpu.VMEM((16, 1), jnp.float32),
            pltpu.VMEM((16, 1), jnp.float32),
            pltpu.VMEM((2, 16, 1), jnp.float32),
            pltpu.VMEM((2, 2, 16, 1), jnp.float32),
            pltpu.VMEM((16, 1), jnp.float32),
            pltpu.VMEM((16, 1), jnp.float32),
            pltpu.SMEM((1,), jnp.int32),
            pltpu.SMEM((1,), jnp.int32),
            pltpu.SemaphoreType.REGULAR,
        ],
        compiler_params=pltpu.CompilerParams(
            vmem_limit_bytes=100 * 1024 * 1024,
        ),
    )

    outT, velT, _ = fn(xyzT, featT, mask, W, g, b)
    return (xyz, outT.T, offset, velT.T)
